# Initial kernel scaffold; baseline (speedup 1.0000x reference)
import jax
import jax.numpy as jnp
from jax.experimental import pallas as pl


def kernel(x, token_table, pos_table):
    raise NotImplementedError("write your pallas kernel here")


# speedup vs baseline: 3.3571x; 3.3571x over previous
import functools

import jax
import jax.numpy as jnp
from jax import lax
from jax.experimental import pallas as pl
from jax.experimental.pallas import tpu as pltpu
from jax.experimental.pallas import tpu_sc as plsc

LANES = 16
NC, NS = 2, 16
NW = NC * NS

MAXLEN = 512
EMBED = 64
GATHER_W = 128


def _tpe_body(idx_hbm, tok_hbm, pos_hbm, out_hbm, idx_v, rows_v, pos_v, sem,
              *, per_w, nchunk, ngather):
    wid = lax.axis_index("s") * NC + lax.axis_index("c")
    base_blk = wid * (per_w // GATHER_W)
    base_row = wid * per_w

    pltpu.sync_copy(pos_hbm, pos_v)

    @pl.loop(0, nchunk)
    def _chunk(c):
        blk0 = base_blk + c * ngather
        row0 = base_row + c * MAXLEN
        pltpu.sync_copy(idx_hbm.at[pl.ds(blk0, ngather)], idx_v)

        copies = [
            pltpu.async_copy(
                tok_hbm.at[idx_v.at[j]],
                rows_v.at[pl.ds(j * GATHER_W, GATHER_W)],
                sem,
            )
            for j in range(ngather)
        ]
        for cp in copies:
            cp.wait()

        @pl.loop(0, MAXLEN, step=8)
        def _add(r):
            for dr in range(8):
                for cc in range(0, EMBED, LANES):
                    v = pos_v[r + dr, pl.ds(cc, LANES)]
                    rows_v[r + dr, pl.ds(cc, LANES)] += v

        pltpu.sync_copy(rows_v, out_hbm.at[pl.ds(row0, MAXLEN)])


def kernel(x, token_table, pos_table):
    batch, seq = x.shape
    vocab, embed = token_table.shape
    total = batch * seq
    per_w = total // NW
    nchunk = per_w // MAXLEN
    ngather = MAXLEN // GATHER_W

    idx = x.reshape(total // GATHER_W, GATHER_W).astype(jnp.int32)
    mesh = plsc.VectorSubcoreMesh(core_axis_name="c", subcore_axis_name="s")

    run = pl.kernel(
        functools.partial(_tpe_body, per_w=per_w, nchunk=nchunk,
                          ngather=ngather),
        out_type=jax.ShapeDtypeStruct((total, embed), jnp.float32),
        mesh=mesh,
        scratch_types=[
            pltpu.VMEM((ngather, GATHER_W), jnp.int32),
            pltpu.VMEM((MAXLEN, embed), jnp.float32),
            pltpu.VMEM((MAXLEN, embed), jnp.float32),
            pltpu.SemaphoreType.DMA,
        ],
        compiler_params=pltpu.CompilerParams(use_tc_tiling_on_sc=False),
    )
    out = run(idx, token_table, pos_table)
    return out.reshape(batch, seq, embed)


# speedup vs baseline: 4.0689x; 1.2120x over previous
import functools

import jax
import jax.numpy as jnp
from jax import lax
from jax.experimental import pallas as pl
from jax.experimental.pallas import tpu as pltpu
from jax.experimental.pallas import tpu_sc as plsc

LANES = 16
NC, NS = 2, 16
NW = NC * NS

MAXLEN = 512
EMBED = 64
GATHER_W = 128
CHUNK = 256
NBUF = 4
NG = CHUNK // GATHER_W


def _tpe_body(idx_hbm, tok_hbm, pos_hbm, out_hbm, idx_v, pos_v, rows, g_sems,
              o_sems, ld_sem, *, per_w, nchunk):
    wid = lax.axis_index("s") * NC + lax.axis_index("c")
    blk_per_w = per_w // GATHER_W
    base_row = wid * per_w

    pltpu.async_copy(idx_hbm.at[pl.ds(wid * blk_per_w, blk_per_w)], idx_v,
                     ld_sem)
    pltpu.make_async_copy(idx_hbm.at[pl.ds(0, blk_per_w)], idx_v,
                          ld_sem).wait()
    pltpu.async_copy(pos_hbm, pos_v, ld_sem)
    pltpu.make_async_copy(pos_hbm, pos_v, ld_sem).wait()

    def fire_gathers(c, buf):
        for j in range(NG):
            pltpu.async_copy(
                tok_hbm.at[idx_v.at[c * NG + j]],
                rows[buf].at[pl.ds(j * GATHER_W, GATHER_W)],
                g_sems[buf],
            )

    def wait_gathers(c, buf):
        for j in range(NG):
            pltpu.make_async_copy(
                tok_hbm.at[idx_v.at[c * NG + j]],
                rows[buf].at[pl.ds(j * GATHER_W, GATHER_W)],
                g_sems[buf],
            ).wait()

    def fire_out(c, buf):
        pltpu.async_copy(rows[buf],
                         out_hbm.at[pl.ds(base_row + c * CHUNK, CHUNK)],
                         o_sems[buf])

    def wait_out(c, buf):
        pltpu.make_async_copy(rows[buf],
                              out_hbm.at[pl.ds(base_row + c * CHUNK, CHUNK)],
                              o_sems[buf]).wait()

    fire_gathers(0, 0)

    @pl.loop(0, nchunk, step=NBUF)
    def _quad(c0):
        for k in range(NBUF):
            c = c0 + k
            buf = k
            nxt = (k + 1) % NBUF
            pos_off = (k % 2) * CHUNK

            @pl.when(c + 1 < nchunk)
            def _fire_next(c=c, nxt=nxt):
                @pl.when(c - (NBUF - 1) >= 0)
                def _drain_out(c=c, nxt=nxt):
                    wait_out(c - (NBUF - 1), nxt)

                fire_gathers(c + 1, nxt)

            wait_gathers(c, buf)

            @pl.loop(0, CHUNK, step=8)
            def _add(r, buf=buf, pos_off=pos_off):
                for dr in range(8):
                    for cc in range(0, EMBED, LANES):
                        v = pos_v[pos_off + r + dr, pl.ds(cc, LANES)]
                        rows[buf][r + dr, pl.ds(cc, LANES)] += v

            fire_out(c, buf)

    for k in range(NBUF):
        wait_out(nchunk - NBUF + k, k)


def kernel(x, token_table, pos_table):
    batch, seq = x.shape
    vocab, embed = token_table.shape
    total = batch * seq
    per_w = total // NW
    nchunk = per_w // CHUNK

    idx = x.reshape(total // GATHER_W, GATHER_W).astype(jnp.int32)
    mesh = plsc.VectorSubcoreMesh(core_axis_name="c", subcore_axis_name="s")

    run = pl.kernel(
        functools.partial(_tpe_body, per_w=per_w, nchunk=nchunk),
        out_type=jax.ShapeDtypeStruct((total, embed), jnp.float32),
        mesh=mesh,
        scratch_types=[
            pltpu.VMEM((per_w // GATHER_W, GATHER_W), jnp.int32),
            pltpu.VMEM((MAXLEN, embed), jnp.float32),
            [pltpu.VMEM((CHUNK, embed), jnp.float32) for _ in range(NBUF)],
            [pltpu.SemaphoreType.DMA for _ in range(NBUF)],
            [pltpu.SemaphoreType.DMA for _ in range(NBUF)],
            pltpu.SemaphoreType.DMA,
        ],
        compiler_params=pltpu.CompilerParams(use_tc_tiling_on_sc=False),
    )
    out = run(idx, token_table, pos_table)
    return out.reshape(batch, seq, embed)
